# baseline (speedup 1.0000x reference)
import functools

import jax
import jax.numpy as jnp
from jax import lax
from jax.experimental import pallas as pl
from jax.experimental.pallas import tpu as pltpu
from jax.experimental.pallas import tpu_sc as plsc

_NUM_ROWS = 64
_ROW_STRIDE = 1543
_D = 512
_L = 16
_NW_ACTIVE = _NUM_ROWS // _L


def _make_sc_gather():
    info = plsc.get_sparse_core_info()
    num_cores = info.num_cores

    mesh = plsc.VectorSubcoreMesh(core_axis_name="c", subcore_axis_name="s")

    @functools.partial(
        pl.kernel,
        mesh=mesh,
        out_type=jax.ShapeDtypeStruct((_NUM_ROWS, _D), jnp.float32),
        scratch_types=[
            pltpu.VMEM((_L, _D), jnp.float32),
            pltpu.SemaphoreType.DMA,
        ],
    )
    def sc_gather(table_hbm, out_hbm, rows_v, sem):
        wid = lax.axis_index("s") * num_cores + lax.axis_index("c")

        @pl.when(wid < _NW_ACTIVE)
        def _():
            idx = (lax.iota(jnp.int32, _L) + wid * _L) * _ROW_STRIDE
            pltpu.async_copy(table_hbm.at[idx], rows_v, sem).wait()
            pltpu.sync_copy(rows_v, out_hbm.at[pl.ds(wid * _L, _L)])

    return sc_gather


_sc_gather = _make_sc_gather()


def kernel(inputs):
    return _sc_gather(inputs)


# speedup vs baseline: 1.0711x; 1.0711x over previous
import functools

import jax
import jax.numpy as jnp
from jax import lax
from jax.experimental import pallas as pl
from jax.experimental.pallas import tpu as pltpu
from jax.experimental.pallas import tpu_sc as plsc

_NUM_ROWS = 64
_ROW_STRIDE = 1543
_D = 512
_L = 16
_NW_ACTIVE = _NUM_ROWS // _L


def _make_sc_gather():
    num_cores = 1

    mesh = plsc.VectorSubcoreMesh(
        core_axis_name="c", subcore_axis_name="s", num_cores=num_cores
    )

    @functools.partial(
        pl.kernel,
        mesh=mesh,
        out_type=jax.ShapeDtypeStruct((_NUM_ROWS, _D), jnp.float32),
        scratch_types=[
            pltpu.VMEM((_L, _D), jnp.float32),
            pltpu.SemaphoreType.DMA,
        ],
    )
    def sc_gather(table_hbm, out_hbm, rows_v, sem):
        wid = lax.axis_index("s") * num_cores + lax.axis_index("c")

        @pl.when(wid < _NW_ACTIVE)
        def _():
            idx = (lax.iota(jnp.int32, _L) + wid * _L) * _ROW_STRIDE
            pltpu.async_copy(table_hbm.at[idx], rows_v, sem).wait()
            pltpu.sync_copy(rows_v, out_hbm.at[pl.ds(wid * _L, _L)])

    return sc_gather


_sc_gather = _make_sc_gather()


def kernel(inputs):
    return _sc_gather(inputs)
